# baseline (speedup 1.0000x reference)
import functools

import jax
import jax.numpy as jnp
from jax import lax
from jax.experimental import pallas as pl
from jax.experimental.pallas import tpu as pltpu
from jax.experimental.pallas import tpu_sc as plsc

_D = 64
_NW = 32
_CHUNK = 128


@functools.lru_cache(maxsize=None)
def _build(B: int):
    bpw = B // _NW
    nck = bpw // _CHUNK
    mesh = plsc.VectorSubcoreMesh(core_axis_name="c", subcore_axis_name="s")

    @functools.partial(
        pl.kernel,
        mesh=mesh,
        out_type=jax.ShapeDtypeStruct((B, _D), jnp.float32),
        scratch_types=[
            pltpu.VMEM((nck, _CHUNK), jnp.int32),
            pltpu.VMEM((_CHUNK, _D), jnp.float32),
            pltpu.SemaphoreType.DMA,
        ],
        compiler_params=pltpu.CompilerParams(use_tc_tiling_on_sc=False),
    )
    def gather_kernel(idx_hbm, tab_hbm, out_hbm, idx_v, rows_v, sem):
        wid = lax.axis_index("s") * 2 + lax.axis_index("c")
        pltpu.sync_copy(idx_hbm.at[pl.ds(wid * nck, nck)], idx_v)
        base = wid * bpw

        def body(j, carry):
            pltpu.async_copy(tab_hbm.at[idx_v.at[j]], rows_v, sem).wait()
            pltpu.sync_copy(rows_v, out_hbm.at[pl.ds(base + j * _CHUNK, _CHUNK)])
            return carry

        lax.fori_loop(0, nck, body, 0)

    return gather_kernel


def kernel(token_ids, weights):
    B = token_ids.shape[0] * token_ids.shape[1]
    flat = token_ids.reshape(B // _CHUNK, _CHUNK).astype(jnp.int32)
    out = _build(B)(flat, weights)
    return out.reshape(token_ids.shape + (weights.shape[1],))


# speedup vs baseline: 1.0602x; 1.0602x over previous
import functools

import jax
import jax.numpy as jnp
from jax import lax
from jax.experimental import pallas as pl
from jax.experimental.pallas import tpu as pltpu
from jax.experimental.pallas import tpu_sc as plsc

_D = 64
_NW = 32
_CHUNK = 128


_NBUF = 8


@functools.lru_cache(maxsize=None)
def _build(B: int):
    bpw = B // _NW
    nck = bpw // _CHUNK
    ngrp = nck // _NBUF
    assert ngrp * _NBUF == nck
    mesh = plsc.VectorSubcoreMesh(core_axis_name="c", subcore_axis_name="s")

    scratch = [pltpu.VMEM((nck, _CHUNK), jnp.int32)]
    scratch += [pltpu.VMEM((_CHUNK, _D), jnp.float32) for _ in range(_NBUF)]
    scratch += [pltpu.SemaphoreType.DMA for _ in range(2 * _NBUF)]

    @functools.partial(
        pl.kernel,
        mesh=mesh,
        out_type=jax.ShapeDtypeStruct((B, _D), jnp.float32),
        scratch_types=scratch,
        compiler_params=pltpu.CompilerParams(use_tc_tiling_on_sc=False),
    )
    def gather_kernel(idx_hbm, tab_hbm, out_hbm, idx_v, *rest):
        bufs = rest[:_NBUF]
        gsem = rest[_NBUF:2 * _NBUF]
        ssem = rest[2 * _NBUF:]
        wid = lax.axis_index("s") * 2 + lax.axis_index("c")
        pltpu.sync_copy(idx_hbm.at[pl.ds(wid * nck, nck)], idx_v)
        base = wid * bpw

        def out_ref(j):
            return out_hbm.at[pl.ds(base + j * _CHUNK, _CHUNK)]

        for b in range(_NBUF):
            pltpu.async_copy(tab_hbm.at[idx_v.at[b]], bufs[b], gsem[b])

        def group(g, carry):
            for b in range(_NBUF):
                j = g * _NBUF + b
                pltpu.make_async_copy(tab_hbm.at[idx_v.at[j]], bufs[b],
                                      gsem[b]).wait()
                pltpu.async_copy(bufs[b], out_ref(j), ssem[b])
            for b in range(_NBUF):
                j = g * _NBUF + b
                pltpu.make_async_copy(bufs[b], out_ref(j), ssem[b]).wait()
                pltpu.async_copy(tab_hbm.at[idx_v.at[j + _NBUF]], bufs[b],
                                 gsem[b])
            return carry

        lax.fori_loop(0, ngrp - 1, group, 0)

        for b in range(_NBUF):
            j = (ngrp - 1) * _NBUF + b
            pltpu.make_async_copy(tab_hbm.at[idx_v.at[j]], bufs[b],
                                  gsem[b]).wait()
            pltpu.async_copy(bufs[b], out_ref(j), ssem[b])
        for b in range(_NBUF):
            j = (ngrp - 1) * _NBUF + b
            pltpu.make_async_copy(bufs[b], out_ref(j), ssem[b]).wait()

    return gather_kernel


def kernel(token_ids, weights):
    B = token_ids.shape[0] * token_ids.shape[1]
    flat = token_ids.reshape(B // _CHUNK, _CHUNK).astype(jnp.int32)
    out = _build(B)(flat, weights)
    return out.reshape(token_ids.shape + (weights.shape[1],))


# speedup vs baseline: 1.0899x; 1.0279x over previous
import functools

import jax
import jax.numpy as jnp
from jax import lax
from jax.experimental import pallas as pl
from jax.experimental.pallas import tpu as pltpu
from jax.experimental.pallas import tpu_sc as plsc

_D = 64
_NW = 32
_CHUNK = 128


@functools.lru_cache(maxsize=None)
def _build(T: int, B: int):
    bpw = B // _NW
    nck = bpw // _CHUNK
    total = T * nck
    mesh = plsc.VectorSubcoreMesh(core_axis_name="c", subcore_axis_name="s")

    scratch = dict(
        tidb=pltpu.VMEM((bpw,), jnp.int32),
        ridxs=[pltpu.VMEM((_CHUNK,), jnp.int32) for _ in range(2)],
        hoffs=[pltpu.VMEM((_CHUNK,), jnp.int32) for _ in range(2)],
        gbufs=[pltpu.VMEM((_CHUNK, 128), jnp.float32) for _ in range(2)],
        obufs=[pltpu.VMEM((_D, _CHUNK), jnp.float32) for _ in range(2)],
        gsems=[pltpu.SemaphoreType.DMA for _ in range(2)],
        osems=[pltpu.SemaphoreType.DMA for _ in range(2)],
    )

    @functools.partial(
        pl.kernel,
        mesh=mesh,
        out_type=jax.ShapeDtypeStruct((T, _D, B), jnp.float32),
        scratch_types=scratch,
        compiler_params=pltpu.CompilerParams(needs_layout_passes=False),
    )
    def gather_kernel(tid_hbm, tab_hbm, out_hbm, *, tidb, ridxs, hoffs,
                      gbufs, obufs, gsems, osems):
        wid = lax.axis_index("s") * 2 + lax.axis_index("c")
        b0w = wid * bpw
        lane = lax.iota(jnp.int32, 16)

        def stage_tid(t):
            pltpu.sync_copy(tid_hbm.at[t, pl.ds(b0w, bpw)], tidb)

        def prep_chunk(slot, k):
            for v in range(_CHUNK // 16):
                tv = tidb[pl.ds(k * _CHUNK + v * 16, 16)]
                ridxs[slot][pl.ds(v * 16, 16)] = lax.shift_right_logical(tv, 1)
                hoffs[slot][pl.ds(v * 16, 16)] = (tv & 1) * _D

        def start_gather(slot):
            pltpu.async_copy(tab_hbm.at[ridxs[slot]], gbufs[slot], gsems[slot])

        def wait_gather(slot):
            pltpu.make_async_copy(tab_hbm.at[ridxs[slot]], gbufs[slot],
                                  gsems[slot]).wait()

        def transpose_chunk(slot):
            gbuf, obuf = gbufs[slot], obufs[slot]

            def lgroup(g, carry):
                l_ids = g * 16 + lane
                hv = hoffs[slot][pl.ds(g * 16, 16)]
                for d in range(16):
                    rot = (lane + d) & 15
                    for cg in range(_D // 16):
                        cvec = cg * 16 + rot
                        vals = plsc.load_gather(gbuf, [l_ids, hv + cvec])
                        plsc.store_scatter(obuf, [cvec, l_ids], vals)
                return carry

            lax.fori_loop(0, _CHUNK // 16, lgroup, 0)

        def out_ref(t, k):
            return out_hbm.at[t, :, pl.ds(b0w + k * _CHUNK, _CHUNK)]

        def start_store(slot, t, k):
            pltpu.async_copy(obufs[slot], out_ref(t, k), osems[slot])

        def wait_store(slot, t, k):
            pltpu.make_async_copy(obufs[slot], out_ref(t, k),
                                  osems[slot]).wait()

        stage_tid(0)
        prep_chunk(0, 0)
        start_gather(0)

        def step(j, slot):
            t = j // nck
            k = j - t * nck
            jn = j + 1
            tn = jn // nck
            kn = jn - tn * nck
            nslot = 1 - slot

            @pl.when(jn < total)
            def _():
                @pl.when(kn == 0)
                def _():
                    stage_tid(tn)
                prep_chunk(nslot, kn)
                @pl.when(jn >= 2)
                def _():
                    jp = jn - 2
                    tp = jp // nck
                    wait_store(nslot, tp, jp - tp * nck)
                start_gather(nslot)

            wait_gather(slot)
            transpose_chunk(slot)
            start_store(slot, t, k)

        def group(g, carry):
            for b in range(2):
                step(2 * g + b, b)
            return carry

        lax.fori_loop(0, total // 2, group, 0)
        for back in (2, 1):
            j = total - back
            t = j // nck
            wait_store(j & 1, t, j - t * nck)

    return gather_kernel


def kernel(token_ids, weights):
    Bt, T = token_ids.shape
    B = Bt
    tid_t = token_ids.T.astype(jnp.int32)
    tab = weights.reshape(weights.shape[0] // 2, 2 * weights.shape[1])
    out_t = _build(T, B)(tid_t, tab)
    return jnp.transpose(out_t, (2, 0, 1))
